# Initial kernel scaffold; baseline (speedup 1.0000x reference)
import jax
import jax.numpy as jnp
from jax.experimental import pallas as pl


def kernel(x, Ws, als, ars, bs, ei_u, ei_o, ei_near, ei_same):
    raise NotImplementedError("write your pallas kernel here")


# speedup vs baseline: 8.9438x; 8.9438x over previous
import functools

import jax
import jax.numpy as jnp
from jax.experimental import pallas as pl

_N = 50000
_IN = 128
_HEADS = 4
_DH = 32
_R = 4
_BN = 1000


def _proj_kernel(x_ref, w_ref, albd_ref, arbd_ref, feat_ref, el_ref, er_ref):
    f = jnp.dot(x_ref[...], w_ref[0], preferred_element_type=jnp.float32)
    feat_ref[0] = f
    el_ref[0] = jnp.dot(f, albd_ref[0], preferred_element_type=jnp.float32)
    er_ref[0] = jnp.dot(f, arbd_ref[0], preferred_element_type=jnp.float32)


def _project(h, W_l, albd_l, arbd_l):
    grid = (_R, _N // _BN)
    return pl.pallas_call(
        _proj_kernel,
        grid=grid,
        in_specs=[
            pl.BlockSpec((_BN, _IN), lambda r, i: (i, 0)),
            pl.BlockSpec((1, _IN, _HEADS * _DH), lambda r, i: (r, 0, 0)),
            pl.BlockSpec((1, _IN, _HEADS), lambda r, i: (r, 0, 0)),
            pl.BlockSpec((1, _IN, _HEADS), lambda r, i: (r, 0, 0)),
        ],
        out_specs=[
            pl.BlockSpec((1, _BN, _HEADS * _DH), lambda r, i: (r, i, 0)),
            pl.BlockSpec((1, _BN, _HEADS), lambda r, i: (r, i, 0)),
            pl.BlockSpec((1, _BN, _HEADS), lambda r, i: (r, i, 0)),
        ],
        out_shape=[
            jax.ShapeDtypeStruct((_R, _N, _HEADS * _DH), jnp.float32),
            jax.ShapeDtypeStruct((_R, _N, _HEADS), jnp.float32),
            jax.ShapeDtypeStruct((_R, _N, _HEADS), jnp.float32),
        ],
    )(h, W_l, albd_l, arbd_l)


def _combine_kernel(seg_ref, h_ref, bsum_ref, o_ref, *, act):
    v = seg_ref[...] + 4.0 * h_ref[...] + bsum_ref[...]
    if act:
        v = jnp.where(v > 0.0, v, jnp.exp(jnp.minimum(v, 0.0)) - 1.0)
    o_ref[...] = v


def _combine(segsum, h, bsum, act):
    return pl.pallas_call(
        functools.partial(_combine_kernel, act=act),
        grid=(_N // _BN,),
        in_specs=[
            pl.BlockSpec((_BN, _IN), lambda i: (i, 0)),
            pl.BlockSpec((_BN, _IN), lambda i: (i, 0)),
            pl.BlockSpec((1, _IN), lambda i: (0, 0)),
        ],
        out_specs=pl.BlockSpec((_BN, _IN), lambda i: (i, 0)),
        out_shape=jax.ShapeDtypeStruct((_N, _IN), jnp.float32),
    )(segsum, h, bsum)


def _blockdiag(a_l):
    eye = jnp.eye(_HEADS, dtype=jnp.float32)
    bd = a_l[:, :, :, None] * eye[:, None, :]
    return bd.reshape(_R, _IN, _HEADS)


def _layer(h, W_l, albd_l, arbd_l, bsum_l, edges, act):
    feat, el, er = _project(h, W_l, albd_l, arbd_l)
    segsum = jnp.zeros((_N, _IN), dtype=jnp.float32)
    for r in range(_R):
        src = edges[r][0]
        dst = edges[r][1]
        e = el[r][src] + er[r][dst]
        e = jnp.where(e > 0.0, e, 0.2 * e)
        emax = jax.ops.segment_max(e, dst, num_segments=_N)
        a = jnp.exp(e - emax[dst])
        denom = jax.ops.segment_sum(a, dst, num_segments=_N)
        alpha = a / (denom[dst] + 1e-9)
        alpha_full = jnp.repeat(alpha, _DH, axis=1)
        msg = feat[r][src] * alpha_full
        segsum = segsum + jax.ops.segment_sum(msg, dst, num_segments=_N)
    return _combine(segsum, h, bsum_l, act)


def kernel(x, Ws, als, ars, bs, ei_u, ei_o, ei_near, ei_same):
    edges = [ei_u, ei_o, ei_near, ei_same]
    albd = jax.vmap(_blockdiag)(als)
    arbd = jax.vmap(_blockdiag)(ars)
    bsum = jnp.sum(bs, axis=1)[:, None, :]
    h = _layer(x, Ws[0], albd[0], arbd[0], bsum[0], edges, act=True)
    out = _layer(h, Ws[1], albd[1], arbd[1], bsum[1], edges, act=False)
    return out
